# baseline (speedup 1.0000x reference)
import jax
import jax.numpy as jnp
from jax import lax
from jax.experimental import pallas as pl
from jax.experimental.pallas import tpu as pltpu

_B, _G, _E = 4, 64, 8
_DS, _DE, _H, _A = 1024, 64, 1024, 16
_DIN = _DS + _DE
_N = _B * _G
_T = 64
_NCHUNK = _N // _T


def _moe_kernel(logits_ref, gum_ref, state_ref, emb_ref, bcat_ref,
                w3_ref, w1_any, w2_any, out_ref,
                permbuf, counts_vm, counts_sm, csem):
    bf = jnp.bfloat16
    f32 = jnp.float32

    out_ref[...] = jnp.zeros_like(out_ref)
    logits = logits_ref[...]
    scores = (jnp.concatenate([logits] * _B, axis=0)
              + gum_ref[...])
    best = scores[:, 0:1]
    beste = jnp.zeros((_N, 1), jnp.int32)
    for ee in range(1, _E):
        s = scores[:, ee:ee + 1]
        m = s > best
        beste = jnp.where(m, ee, beste)
        best = jnp.maximum(best, s)
    oh = (beste == lax.broadcasted_iota(jnp.int32, (_N, _E), 1)
          ).astype(bf)
    ones_row = jnp.ones((1, _N), bf)
    counts_row = jnp.dot(ones_row, oh, preferred_element_type=f32)
    counts_vm[...] = counts_row.astype(jnp.int32)
    ti = lax.broadcasted_iota(jnp.int32, (_N, _N), 0)
    tj = lax.broadcasted_iota(jnp.int32, (_N, _N), 1)
    tril = (tj <= ti).astype(bf)
    csum = jnp.dot(tril, oh, preferred_element_type=f32)
    slot = (jnp.sum(csum * oh.astype(f32), axis=1, keepdims=True)
            - 1.0)
    ssel = (slot == lax.broadcasted_iota(
        jnp.int32, (_N, _N), 1).astype(f32)).astype(bf)
    tvals = (lax.broadcasted_iota(jnp.int32, (_N, _E), 0).astype(bf)
             * oh)
    permf = lax.dot_general(ssel, tvals, (((0,), (0,)), ((), ())),
                            preferred_element_type=f32)
    permbuf[...] = permf.astype(jnp.int32)
    pltpu.make_async_copy(counts_vm, counts_sm, csem).start()
    pltpu.make_async_copy(counts_vm, counts_sm, csem).wait()

    state_bf = state_ref[...].astype(bf)
    emb_bf = emb_ref[...].astype(bf)

    def expert_body(idxs, w1_ref, w2_ref):
        e = idxs[0]
        count = counts_sm[0, e]
        w1 = w1_ref[0]
        sp = jnp.dot(state_bf, w1[_DE:, :].astype(bf),
                     preferred_element_type=f32)
        ep = jnp.dot(emb_bf, w1[:_DE, :].astype(bf),
                     preferred_element_type=f32)
        w2b = w2_ref[0].astype(bf)
        w3b = w3_ref[pl.ds(e, 1)][0].astype(bf)
        brow = bcat_ref[pl.ds(e, 1), :]
        b1v = brow[:, :_H]
        b2v = brow[:, _H:2 * _H]
        b3v = brow[:, 2 * _H:]

        def chunk(j):
            tid8 = permbuf[pl.ds(j * _T, _T), :]
            lane = lax.broadcasted_iota(jnp.int32, (_T, _E), 1)
            tid = jnp.sum(jnp.where(lane == e, tid8, 0), axis=1,
                          keepdims=True)
            riota = lax.broadcasted_iota(jnp.int32, (_T, 1), 0)
            valid = (j * _T + riota) < count
            bidx = tid // _G
            gidx = tid - bidx * _G
            oh_b = (bidx == lax.broadcasted_iota(jnp.int32, (_T, _B), 1)
                    ).astype(bf)
            oh_g = (gidx == lax.broadcasted_iota(jnp.int32, (_T, _G), 1)
                    ).astype(bf)
            h1c = jnp.maximum(
                jnp.dot(oh_b, sp.astype(bf), preferred_element_type=f32)
                + jnp.dot(oh_g, ep.astype(bf), preferred_element_type=f32)
                + b1v, 0.0)
            h2c = jnp.maximum(
                jnp.dot(h1c.astype(bf), w2b, preferred_element_type=f32)
                + b2v, 0.0)
            oc = (jnp.dot(h2c.astype(bf), w3b, preferred_element_type=f32)
                  + b3v)
            oh_t = ((tid == lax.broadcasted_iota(jnp.int32, (_T, _N), 1))
                    & valid).astype(bf)
            out_ref[...] += lax.dot_general(
                oh_t, oc.astype(bf), (((0,), (0,)), ((), ())),
                preferred_element_type=f32)

        chunk(0)
        for j in range(1, _NCHUNK):
            @pl.when(j * _T < count)
            def _(j=j):
                chunk(j)

    pipeline = pltpu.emit_pipeline(
        expert_body,
        grid=(_E,),
        in_specs=[
            pl.BlockSpec((1, _DIN, _H), lambda e: (e, 0, 0)),
            pl.BlockSpec((1, _H, _H), lambda e: (e, 0, 0)),
        ],
        dimension_semantics=(pltpu.PARALLEL,),
        _explicit_indices=True,
    )
    pipeline(w1_any, w2_any)


def _run_moe(logits, gum, state, agent_emb, W1, b1, W2, b2, W3, b3):
    return pl.pallas_call(
        _moe_kernel,
        in_specs=[
            pl.BlockSpec(memory_space=pltpu.VMEM),
            pl.BlockSpec(memory_space=pltpu.VMEM),
            pl.BlockSpec(memory_space=pltpu.VMEM),
            pl.BlockSpec(memory_space=pltpu.VMEM),
            pl.BlockSpec(memory_space=pltpu.VMEM),
            pl.BlockSpec(memory_space=pltpu.VMEM),
            pl.BlockSpec(memory_space=pl.ANY),
            pl.BlockSpec(memory_space=pl.ANY),
        ],
        out_specs=pl.BlockSpec(memory_space=pltpu.VMEM),
        out_shape=jax.ShapeDtypeStruct((_N, _A), jnp.float32),
        scratch_shapes=[
            pltpu.VMEM((_N, _E), jnp.int32),
            pltpu.VMEM((1, _E), jnp.int32),
            pltpu.SMEM((1, _E), jnp.int32),
            pltpu.SemaphoreType.DMA,
        ],
        compiler_params=pltpu.CompilerParams(),
    )(logits, gum, state, agent_emb,
      jnp.concatenate([b1, b2, b3], axis=1), W3, W1, W2)


def kernel(state, assigner_logits, agent_emb, W1, b1, W2, b2, W3, b3):
    u = jax.random.uniform(jax.random.key(1), (_B, _G, _E), jnp.float32,
                           1e-6, 1.0 - 1e-6)
    gum = (-jnp.log(-jnp.log(u))).reshape(_N, _E)
    out = _run_moe(assigner_logits, gum, state, agent_emb, W1, b1, W2, b2,
                   W3, b3)
    return out.reshape(_B, _G, _A)


# speedup vs baseline: 1.0077x; 1.0077x over previous
import jax
import jax.numpy as jnp
from jax import lax
from jax.experimental import pallas as pl
from jax.experimental.pallas import tpu as pltpu

_B, _G, _E = 4, 64, 8
_DS, _DE, _H, _A = 1024, 64, 1024, 16
_DIN = _DS + _DE
_N = _B * _G
_T = 64
_NCHUNK = _N // _T


def _moe_kernel(logits_ref, gum_ref, state_ref, emb_ref, bcat_ref,
                w3_ref, w1_any, w2_any, out_ref,
                permbuf, counts_vm, counts_sm, csem):
    bf = jnp.bfloat16
    f32 = jnp.float32

    out_ref[...] = jnp.zeros_like(out_ref)
    logits = logits_ref[...]
    scores = (jnp.concatenate([logits] * _B, axis=0)
              + gum_ref[...])
    best = scores[:, 0:1]
    beste = jnp.zeros((_N, 1), jnp.int32)
    for ee in range(1, _E):
        s = scores[:, ee:ee + 1]
        m = s > best
        beste = jnp.where(m, ee, beste)
        best = jnp.maximum(best, s)
    oh = (beste == lax.broadcasted_iota(jnp.int32, (_N, _E), 1)
          ).astype(bf)
    ones_row = jnp.ones((1, _N), bf)
    counts_row = jnp.dot(ones_row, oh, preferred_element_type=f32)
    counts_vm[...] = counts_row.astype(jnp.int32)
    ti = lax.broadcasted_iota(jnp.int32, (_N, _N), 0)
    tj = lax.broadcasted_iota(jnp.int32, (_N, _N), 1)
    tril = (tj <= ti).astype(bf)
    csum = jnp.dot(tril, oh, preferred_element_type=f32)
    slot = (jnp.sum(csum * oh.astype(f32), axis=1, keepdims=True)
            - 1.0)
    ssel = (slot == lax.broadcasted_iota(
        jnp.int32, (_N, _N), 1).astype(f32)).astype(bf)
    tvals = (lax.broadcasted_iota(jnp.int32, (_N, _E), 0).astype(bf)
             * oh)
    permf = lax.dot_general(ssel, tvals, (((0,), (0,)), ((), ())),
                            preferred_element_type=f32)
    permbuf[...] = permf.astype(jnp.int32)
    pltpu.make_async_copy(counts_vm, counts_sm, csem).start()
    pltpu.make_async_copy(counts_vm, counts_sm, csem).wait()

    state_bf = state_ref[...].astype(bf)
    emb_bf = emb_ref[...].astype(bf)

    def expert_body(idxs, w1_ref, w2_ref):
        e = idxs[0]
        count = counts_sm[0, e]
        w1 = w1_ref[0]
        sp = jnp.dot(state_bf, w1[_DE:, :].astype(bf),
                     preferred_element_type=f32)
        ep = jnp.dot(emb_bf, w1[:_DE, :].astype(bf),
                     preferred_element_type=f32)
        w2b = w2_ref[0].astype(bf)
        w3b = w3_ref[pl.ds(e, 1)][0].astype(bf)
        brow = bcat_ref[pl.ds(e, 1), :]
        b1v = brow[:, :_H]
        b2v = brow[:, _H:2 * _H]
        b3v = brow[:, 2 * _H:]

        def chunk(j):
            tid8 = permbuf[pl.ds(j * _T, _T), :]
            lane = lax.broadcasted_iota(jnp.int32, (_T, _E), 1)
            tid = jnp.sum(jnp.where(lane == e, tid8, 0), axis=1,
                          keepdims=True)
            riota = lax.broadcasted_iota(jnp.int32, (_T, 1), 0)
            valid = (j * _T + riota) < count
            bidx = tid // _G
            gidx = tid - bidx * _G
            oh_b = (bidx == lax.broadcasted_iota(jnp.int32, (_T, _B), 1)
                    ).astype(bf)
            oh_g = (gidx == lax.broadcasted_iota(jnp.int32, (_T, _G), 1)
                    ).astype(bf)
            h1c = jnp.maximum(
                jnp.dot(oh_b, sp.astype(bf), preferred_element_type=f32)
                + jnp.dot(oh_g, ep.astype(bf), preferred_element_type=f32)
                + b1v, 0.0)
            h2c = jnp.maximum(
                jnp.dot(h1c.astype(bf), w2b, preferred_element_type=f32)
                + b2v, 0.0)
            oc = (jnp.dot(h2c.astype(bf), w3b, preferred_element_type=f32)
                  + b3v)
            oh_t = ((tid == lax.broadcasted_iota(jnp.int32, (_T, _N), 1))
                    & valid).astype(bf)
            out_ref[...] += lax.dot_general(
                oh_t, oc.astype(bf), (((0,), (0,)), ((), ())),
                preferred_element_type=f32)

        chunk(0)
        for j in range(1, _NCHUNK):
            @pl.when(j * _T < count)
            def _(j=j):
                chunk(j)

    pipeline = pltpu.emit_pipeline(
        expert_body,
        grid=(_E,),
        in_specs=[
            pl.BlockSpec((1, _DIN, _H), lambda e: (e, 0, 0)),
            pl.BlockSpec((1, _H, _H), lambda e: (e, 0, 0)),
        ],
        _explicit_indices=True,
    )
    pipeline(w1_any, w2_any)


def _run_moe(logits, gum, state, agent_emb, W1, b1, W2, b2, W3, b3):
    return pl.pallas_call(
        _moe_kernel,
        in_specs=[
            pl.BlockSpec(memory_space=pltpu.VMEM),
            pl.BlockSpec(memory_space=pltpu.VMEM),
            pl.BlockSpec(memory_space=pltpu.VMEM),
            pl.BlockSpec(memory_space=pltpu.VMEM),
            pl.BlockSpec(memory_space=pltpu.VMEM),
            pl.BlockSpec(memory_space=pltpu.VMEM),
            pl.BlockSpec(memory_space=pl.ANY),
            pl.BlockSpec(memory_space=pl.ANY),
        ],
        out_specs=pl.BlockSpec(memory_space=pltpu.VMEM),
        out_shape=jax.ShapeDtypeStruct((_N, _A), jnp.float32),
        scratch_shapes=[
            pltpu.VMEM((_N, _E), jnp.int32),
            pltpu.VMEM((1, _E), jnp.int32),
            pltpu.SMEM((1, _E), jnp.int32),
            pltpu.SemaphoreType.DMA,
        ],
        compiler_params=pltpu.CompilerParams(),
    )(logits, gum, state, agent_emb,
      jnp.concatenate([b1, b2, b3], axis=1), W3, W1, W2)


def kernel(state, assigner_logits, agent_emb, W1, b1, W2, b2, W3, b3):
    u = jax.random.uniform(jax.random.key(1), (_B, _G, _E), jnp.float32,
                           1e-6, 1.0 - 1e-6)
    gum = (-jnp.log(-jnp.log(u))).reshape(_N, _E)
    out = _run_moe(assigner_logits, gum, state, agent_emb, W1, b1, W2, b2,
                   W3, b3)
    return out.reshape(_B, _G, _A)
